# baseline (speedup 1.0000x reference)
import functools

import jax
import jax.numpy as jnp
import numpy as np
from jax.experimental import pallas as pl
from jax.experimental.pallas import tpu as pltpu

_B, _IN, _MID, _NC, _K = 128, 2048, 8192, 2048, 16
_TAU = 0.5
_EPS = 1e-5

_BM1 = 512
_BM2 = 256
_BM3 = 128
_NA = _MID // _BM1
_NB = _MID // _BM2
_NC3 = _NC // _BM3
_NSTEPS = _NA + _NB + _NC3


@functools.lru_cache(maxsize=1)
def _noise_consts():
    with jax.ensure_compile_time_eval():
        return _noise_consts_impl()


def _noise_consts_impl():
    dk = jax.random.key(42)
    keep0 = jax.random.bernoulli(jax.random.fold_in(dk, 0), 0.5, (_B, _IN))
    keep1 = jax.random.bernoulli(jax.random.fold_in(dk, 1), 0.5, (_B, _MID))
    gk = jax.random.key(7)
    g = jnp.stack([
        jax.random.gumbel(jax.random.fold_in(gk, i), (_B, _NC), jnp.float32)
        for i in range(_K)
    ])
    m0 = keep0.astype(jnp.int8) * 2
    m1 = keep1.astype(jnp.int8) * 2
    return (np.asarray(m0), np.asarray(m1), np.asarray(g))


def _bn_relu(acc, gamma, beta):
    mu = jnp.mean(acc, axis=0, keepdims=True)
    var = jnp.mean((acc - mu) ** 2, axis=0, keepdims=True)
    y = (acc - mu) / jnp.sqrt(var + _EPS) * gamma + beta
    return jnp.maximum(y, 0.0)


def _fused_kernel(f_ref, m0_ref, w1_ref, b1_ref, g1_ref, be1_ref, m1_ref,
                  w2_ref, b2_ref, g2_ref, be2_ref, w3_ref, b3_ref, g_ref,
                  z_ref, x_ref, h1_ref, h2_ref, mask_ref):
    t = pl.program_id(0)

    @pl.when(t == 0)
    def _dropout_in():
        x_ref[...] = f_ref[...] * m0_ref[...].astype(jnp.float32)

    @pl.when(t < _NA)
    def _layer1():
        acc = jax.lax.dot_general(
            x_ref[...], w1_ref[...], (((1,), (1,)), ((), ())),
            preferred_element_type=jnp.float32)
        y = _bn_relu(acc + b1_ref[...], g1_ref[...], be1_ref[...])
        h1_ref[:, pl.ds(t * _BM1, _BM1)] = y * m1_ref[...].astype(jnp.float32)

    @pl.when(jnp.logical_and(t >= _NA, t < _NA + _NB))
    def _layer2():
        j = t - _NA
        acc = jax.lax.dot_general(
            h1_ref[...], w2_ref[...], (((1,), (1,)), ((), ())),
            preferred_element_type=jnp.float32)
        y = _bn_relu(acc + b2_ref[...], g2_ref[...], be2_ref[...])
        h2_ref[:, pl.ds(j * _BM2, _BM2)] = y

    @pl.when(t >= _NA + _NB)
    def _layer3():
        j = t - _NA - _NB
        acc = jax.lax.dot_general(
            h2_ref[...], w3_ref[...], (((1,), (1,)), ((), ())),
            preferred_element_type=jnp.float32)
        mask_ref[:, pl.ds(j * _BM3, _BM3)] = acc + b3_ref[...]

    @pl.when(t == _NSTEPS - 1)
    def _select():
        col = jax.lax.broadcasted_iota(jnp.int32, (_B, _NC), 1)
        neg_inf = jnp.float32(-jnp.inf)
        z_ref[...] = jnp.zeros((_B, _NC), jnp.float32)

        def body(k, carry):
            rem = mask_ref[...]
            logits = (rem + g_ref[k]) / _TAU
            m = jnp.max(logits, axis=1, keepdims=True)
            e = jnp.exp(logits - m)
            s = jnp.sum(e, axis=1, keepdims=True)
            z_ref[...] = jnp.maximum(z_ref[...], e / s)
            idx = jnp.min(jnp.where(logits == m, col, _NC), axis=1,
                          keepdims=True)
            mask_ref[...] = jnp.where(col == idx, neg_inf, rem)
            return carry

        jax.lax.fori_loop(0, _K, body, 0)


def kernel(f, W1, b1, g1, be1, W2, b2, g2, be2, W3, b3):
    m0, m1, g = _noise_consts()
    m0 = jnp.asarray(m0)
    m1 = jnp.asarray(m1)
    g = jnp.asarray(g)

    na, nb = _NA, _NB
    i1 = lambda t: (jnp.minimum(t, na - 1), 0)
    c1 = lambda t: (0, jnp.minimum(t, na - 1))
    i2 = lambda t: (jnp.clip(t - na, 0, nb - 1), 0)
    c2 = lambda t: (0, jnp.clip(t - na, 0, nb - 1))
    i3 = lambda t: (jnp.clip(t - na - nb, 0, _NC3 - 1), 0)
    c3 = lambda t: (0, jnp.clip(t - na - nb, 0, _NC3 - 1))
    z = pl.pallas_call(
        _fused_kernel,
        grid=(_NSTEPS,),
        in_specs=[
            pl.BlockSpec((_B, _IN), lambda t: (0, 0)),
            pl.BlockSpec((_B, _IN), lambda t: (0, 0)),
            pl.BlockSpec((_BM1, _IN), i1),
            pl.BlockSpec((1, _BM1), c1),
            pl.BlockSpec((1, _BM1), c1),
            pl.BlockSpec((1, _BM1), c1),
            pl.BlockSpec((_B, _BM1), c1),
            pl.BlockSpec((_BM2, _MID), i2),
            pl.BlockSpec((1, _BM2), c2),
            pl.BlockSpec((1, _BM2), c2),
            pl.BlockSpec((1, _BM2), c2),
            pl.BlockSpec((_BM3, _MID), i3),
            pl.BlockSpec((1, _BM3), c3),
            pl.BlockSpec((_K, _B, _NC), lambda t: (0, 0, 0)),
        ],
        out_specs=pl.BlockSpec((_B, _NC), lambda t: (0, 0)),
        out_shape=jax.ShapeDtypeStruct((_B, _NC), jnp.float32),
        scratch_shapes=[
            pltpu.VMEM((_B, _IN), jnp.float32),
            pltpu.VMEM((_B, _MID), jnp.float32),
            pltpu.VMEM((_B, _MID), jnp.float32),
            pltpu.VMEM((_B, _NC), jnp.float32),
        ],
        compiler_params=pltpu.CompilerParams(
            vmem_limit_bytes=64 * 1024 * 1024),
    )(f, m0, W1, b1.reshape(1, -1), g1.reshape(1, -1), be1.reshape(1, -1),
      m1, W2, b2.reshape(1, -1), g2.reshape(1, -1), be2.reshape(1, -1),
      W3, b3.reshape(1, -1), g)
    return z
